# Initial kernel scaffold; baseline (speedup 1.0000x reference)
import jax
import jax.numpy as jnp
from jax.experimental import pallas as pl


def kernel(in_features, nodes_l0, neigh2, cur1, neigh1, cur2, W0, W1):
    raise NotImplementedError("write your pallas kernel here")


# speedup vs baseline: 7.7005x; 7.7005x over previous
import functools

import jax
import jax.numpy as jnp
from jax import lax
from jax.experimental import pallas as pl
from jax.experimental.pallas import tpu as pltpu
from jax.experimental.pallas import tpu_sc as plsc

NC = 2
NS = 16
NL = 16
NW = NC * NS


def _mesh():
    return plsc.VectorSubcoreMesh(
        core_axis_name="c", subcore_axis_name="s", num_cores=NC, num_subcores=NS
    )


_SC_PARAMS = pltpu.CompilerParams(needs_layout_passes=False)


def _wid():
    return lax.axis_index("s") * NC + lax.axis_index("c")


def _fire(src, dst, sem):
    pltpu.make_async_copy(src, dst, sem).start()


def _drain(src, dst, sem):
    pltpu.make_async_copy(src, dst, sem).wait()


def _compose_kernel(U2, U1, S):
    RPT = U1 // NW
    M = RPT * S

    @functools.partial(
        pl.kernel,
        out_type=(
            jax.ShapeDtypeStruct((U1 * S,), jnp.int32),
            jax.ShapeDtypeStruct((U1,), jnp.int32),
        ),
        mesh=_mesh(),
        compiler_params=_SC_PARAMS,
        scratch_types=[
            pltpu.VMEM((U2,), jnp.int32),
            pltpu.VMEM((M,), jnp.int32),
            pltpu.VMEM((RPT,), jnp.int32),
        ],
    )
    def k(nodes_hbm, neigh2_hbm, cur1_hbm, comp2_hbm, compc_hbm, nodes_v, buf, cbuf):
        base = _wid() * RPT
        pltpu.sync_copy(nodes_hbm, nodes_v)
        pltpu.sync_copy(neigh2_hbm.at[pl.ds(base * S, M)], buf)
        pltpu.sync_copy(cur1_hbm.at[pl.ds(base, RPT)], cbuf)

        def body(i, _):
            off = pl.multiple_of(i * NL, NL)
            v = buf[pl.ds(off, NL)]
            buf[pl.ds(off, NL)] = plsc.load_gather(nodes_v, [v])
            return 0

        lax.fori_loop(0, M // NL, body, 0)

        def cbody(i, _):
            off = pl.multiple_of(i * NL, NL)
            v = cbuf[pl.ds(off, NL)]
            cbuf[pl.ds(off, NL)] = plsc.load_gather(nodes_v, [v])
            return 0

        lax.fori_loop(0, RPT // NL, cbody, 0)

        pltpu.sync_copy(buf, comp2_hbm.at[pl.ds(base * S, M)])
        pltpu.sync_copy(cbuf, compc_hbm.at[pl.ds(base, RPT)])

    return k


def _gather_sum_kernel(D_, U1, S):
    RPT = U1 // NW
    GR = 128
    NG = RPT * S // GR
    RG = GR // S
    FL = 16
    OB = FL * RG
    NX = RPT // GR

    @functools.partial(
        pl.kernel,
        out_type=(
            jax.ShapeDtypeStruct((U1, D_), jnp.float32),
            jax.ShapeDtypeStruct((U1, D_), jnp.float32),
        ),
        mesh=_mesh(),
        compiler_params=_SC_PARAMS,
        scratch_types=[
            pltpu.VMEM((RPT * S,), jnp.int32),
            pltpu.VMEM((RPT,), jnp.int32),
            pltpu.VMEM((2, GR, D_), jnp.float32),
            pltpu.VMEM((OB, D_), jnp.float32),
            pltpu.SemaphoreType.DMA,
            pltpu.SemaphoreType.DMA,
        ],
    )
    def k(feat_hbm, comp2_hbm, compc_hbm, sum_hbm, x_hbm, cidx, ccur, gbuf, obuf,
          semA, semB):
        base = _wid() * RPT
        pltpu.sync_copy(comp2_hbm.at[pl.ds(base * S, RPT * S)], cidx)
        pltpu.sync_copy(compc_hbm.at[pl.ds(base, RPT)], ccur)
        sems = (semA, semB)

        def xsrc(i):
            off = pl.multiple_of(i * GR, GR)
            return feat_hbm.at[ccur.at[pl.ds(off, GR)]]

        _fire(xsrc(0), gbuf.at[0], sems[0])
        for i in range(NX):
            slot = i % 2
            if i + 1 < NX:
                _fire(xsrc(i + 1), gbuf.at[1 - slot], sems[1 - slot])
            _drain(xsrc(i), gbuf.at[slot], sems[slot])
            pltpu.sync_copy(gbuf.at[slot], x_hbm.at[pl.ds(base + i * GR, GR)])

        def gsrc(g):
            off = pl.multiple_of(g * GR, GR)
            return feat_hbm.at[cidx.at[pl.ds(off, GR)]]

        def gstart(g, slot):
            _fire(gsrc(g), gbuf.at[slot], sems[slot])

        gstart(0, 0)

        def pair_body(gp, _):
            for kk in range(2):
                g = gp * 2 + kk
                slot = kk

                @pl.when(g + 1 < NG)
                def _():
                    gstart(g + 1, 1 - slot)

                _drain(gsrc(g), gbuf.at[slot], sems[slot])

                def red_body(r, _):
                    orow = (g % FL) * RG + r
                    for c in range(D_ // NL):
                        cs = pl.ds(c * NL, NL)
                        acc = gbuf[slot, r * S, cs]
                        for s in range(1, S):
                            acc = acc + gbuf[slot, r * S + s, cs]
                        obuf[orow, cs] = acc
                    return 0

                lax.fori_loop(0, RG, red_body, 0)

                @pl.when(g % FL == FL - 1)
                def _():
                    pltpu.sync_copy(
                        obuf, sum_hbm.at[pl.ds(base + (g // FL) * OB, OB)]
                    )
            return 0

        lax.fori_loop(0, NG // 2, pair_body, 0)

    return k


def _gather_sum_small_kernel(H_, B_, S):
    RB = B_ // NW
    M = RB * S
    GR = 128
    NG = M // GR
    RG = GR // S

    @functools.partial(
        pl.kernel,
        out_type=(
            jax.ShapeDtypeStruct((B_, H_), jnp.float32),
            jax.ShapeDtypeStruct((B_, H_), jnp.float32),
        ),
        mesh=_mesh(),
        compiler_params=_SC_PARAMS,
        scratch_types=[
            pltpu.VMEM((M,), jnp.int32),
            pltpu.VMEM((RB,), jnp.int32),
            pltpu.VMEM((2, GR, H_), jnp.float32),
            pltpu.VMEM((RB, H_), jnp.float32),
            pltpu.VMEM((RB, H_), jnp.float32),
            pltpu.SemaphoreType.DMA,
            pltpu.SemaphoreType.DMA,
        ],
    )
    def k(h_hbm, n1_hbm, cur2_hbm, sum_hbm, x_hbm, cidx, ccur, gbuf, obuf, xg,
          semA, semB):
        base = _wid() * RB
        pltpu.sync_copy(n1_hbm.at[pl.ds(base * S, M)], cidx)
        pltpu.sync_copy(cur2_hbm.at[pl.ds(base, RB)], ccur)
        sems = (semA, semB)

        _fire(h_hbm.at[ccur], xg, semA)

        def gsrc(g):
            off = pl.multiple_of(g * GR, GR)
            return h_hbm.at[cidx.at[pl.ds(off, GR)]]

        _fire(gsrc(0), gbuf.at[0], sems[1])
        _drain(h_hbm.at[ccur], xg, semA)
        pltpu.sync_copy(xg, x_hbm.at[pl.ds(base, RB)])

        for g in range(NG):
            slot = g % 2
            if g + 1 < NG:
                _fire(gsrc(g + 1), gbuf.at[1 - slot], sems[g % 2])
            _drain(gsrc(g), gbuf.at[slot], sems[1 - slot])

            def red_body(r, _):
                orow = g * RG + r
                for c in range(H_ // NL):
                    cs = pl.ds(c * NL, NL)
                    acc = gbuf[slot, r * S, cs]
                    for s in range(1, S):
                        acc = acc + gbuf[slot, r * S + s, cs]
                    obuf[orow, cs] = acc
                return 0

            lax.fori_loop(0, RG, red_body, 0)

        pltpu.sync_copy(obuf, sum_hbm.at[pl.ds(base, RB)])

    return k


def _fused_linear(x, s, Wa, Wb, blk):
    R, Dm = x.shape
    Hm = Wa.shape[1]

    def body(x_ref, s_ref, wa_ref, wb_ref, o_ref):
        acc = jnp.dot(x_ref[...], wa_ref[...], preferred_element_type=jnp.float32)
        acc = acc + jnp.dot(s_ref[...], wb_ref[...], preferred_element_type=jnp.float32)
        o_ref[...] = jnp.maximum(acc, 0.0)

    return pl.pallas_call(
        body,
        grid=(R // blk,),
        in_specs=[
            pl.BlockSpec((blk, Dm), lambda i: (i, 0)),
            pl.BlockSpec((blk, Dm), lambda i: (i, 0)),
            pl.BlockSpec((Dm, Hm), lambda i: (0, 0)),
            pl.BlockSpec((Dm, Hm), lambda i: (0, 0)),
        ],
        out_specs=pl.BlockSpec((blk, Hm), lambda i: (i, 0)),
        out_shape=jax.ShapeDtypeStruct((R, Hm), jnp.float32),
    )(x, s, Wa, Wb)


def kernel(in_features, nodes_l0, neigh2, cur1, neigh1, cur2, W0, W1):
    N_, D_ = in_features.shape
    U2 = nodes_l0.shape[0]
    U1, S = neigh2.shape
    B_ = cur2.shape[0]
    H_ = W0.shape[1]

    nodes_l0 = nodes_l0.astype(jnp.int32)
    neigh2f = neigh2.astype(jnp.int32).reshape(U1 * S)
    cur1 = cur1.astype(jnp.int32)
    neigh1f = neigh1.astype(jnp.int32).reshape(B_ * S)
    cur2 = cur2.astype(jnp.int32)

    comp2, compc = _compose_kernel(U2, U1, S)(nodes_l0, neigh2f, cur1)
    sum0, x0 = _gather_sum_kernel(D_, U1, S)(in_features, comp2, compc)
    h1 = _fused_linear(x0, sum0, W0[:D_], W0[D_:] * (1.0 / S), 512)
    sum1, x1 = _gather_sum_small_kernel(H_, B_, S)(h1, neigh1f, cur2)
    out = _fused_linear(x1, sum1, W1[:H_], W1[H_:] * (1.0 / S), 512)
    return out
